# baseline (speedup 1.0000x reference)
import functools

import jax
import jax.numpy as jnp
from jax import lax
from jax.experimental import pallas as pl
from jax.experimental.pallas import tpu as pltpu
from jax.experimental.pallas import tpu_sc as plsc

N = 10000
NP = 10240
D = 128
DH = 256
E = 320000
NC, NS = 2, 16
NW = NC * NS
B = 128
EP = 327680
JB = EP // (NW * B)
NB_ROWS = EP // B
RPS = NP // NS

CH = 16
JB0 = 160
JB1 = 0

def _mesh_fn():
    return plsc.VectorSubcoreMesh(core_axis_name="c", subcore_axis_name="s")


def _f32(*shape):
    return jax.ShapeDtypeStruct(shape, jnp.float32)


def _sc_degree(dst3, ones128, zeros128):
    @functools.partial(
        pl.kernel,
        out_type=_f32(NC * NP, D),
        mesh=_mesh_fn(),
        scratch_types=[
            pltpu.VMEM((JB, B), jnp.int32),
            pltpu.VMEM((B, D), jnp.float32),
            pltpu.VMEM_SHARED((NP, D), jnp.float32),
            pltpu.SemaphoreType.DMA,
        ],
    )
    def k(dst_hbm, ones_hbm, zeros_hbm, out_hbm, idx_v, ones_v, acc_sh, sem):
        cid = lax.axis_index("c")
        sid = lax.axis_index("s")
        w = cid * NS + sid
        pltpu.sync_copy(dst_hbm.at[pl.ds(w * JB, JB)], idx_v)
        pltpu.sync_copy(ones_hbm, ones_v)
        pltpu.sync_copy(zeros_hbm.at[pl.ds(sid * RPS, RPS)],
                        acc_sh.at[pl.ds(sid * RPS, RPS)])
        plsc.subcore_barrier()

        @pl.loop(0, JB)
        def _(j):
            pltpu.sync_copy(ones_v, acc_sh.at[idx_v.at[j]], add=True)

        plsc.subcore_barrier()
        pltpu.sync_copy(acc_sh.at[pl.ds(sid * RPS, RPS)],
                        out_hbm.at[pl.ds(cid * NP + sid * RPS, RPS)])

    return k(dst3, ones128, zeros128)


NBUF = 2


def _sc_aggregate(u, srcb, dstb, zeros128):
    @functools.partial(
        pl.kernel,
        out_type=_f32(NC * NP, D),
        mesh=_mesh_fn(),
        scratch_types=[
            pltpu.VMEM((CH, B), jnp.int32),
            pltpu.VMEM((CH, B), jnp.int32),
        ] + [pltpu.VMEM((B, D), jnp.float32)] * NBUF
          + [pltpu.VMEM_SHARED((NP, D), jnp.float32)]
          + [pltpu.SemaphoreType.DMA] * NBUF,
    )
    def k(u_hbm, src_hbm, dst_hbm, zeros_hbm, out_hbm,
          srcv, dstv, *rest):
        rows = rest[:NBUF]
        acc_sh = rest[NBUF]
        gsem = rest[NBUF + 1:]
        cid = lax.axis_index("c")
        sid = lax.axis_index("s")

        pltpu.sync_copy(zeros_hbm.at[pl.ds(sid * RPS, RPS)],
                        acc_sh.at[pl.ds(sid * RPS, RPS)])
        plsc.subcore_barrier()

        def process_chunk(base):
            pltpu.sync_copy(src_hbm.at[pl.ds(base, CH)], srcv)
            pltpu.sync_copy(dst_hbm.at[pl.ds(base, CH)], dstv)

            for b in range(NBUF):
                pltpu.async_copy(u_hbm.at[srcv.at[b]], rows[b], gsem[b])

            @pl.loop(0, CH - NBUF, step=NBUF)
            def _(g):
                for b in range(NBUF):
                    j = g + b
                    pltpu.make_async_copy(u_hbm.at[srcv.at[j]],
                                          rows[b], gsem[b]).wait()
                    pltpu.sync_copy(rows[b], acc_sh.at[dstv.at[j]], add=True)
                    pltpu.async_copy(u_hbm.at[srcv.at[j + NBUF]],
                                     rows[b], gsem[b])

            for b in range(NBUF):
                j = CH - NBUF + b
                pltpu.make_async_copy(u_hbm.at[srcv.at[j]],
                                      rows[b], gsem[b]).wait()
                pltpu.sync_copy(rows[b], acc_sh.at[dstv.at[j]], add=True)

        @pl.when(cid == 0)
        def _():
            for c in range(JB0 // CH):
                process_chunk(sid * JB0 + c * CH)

        plsc.subcore_barrier()
        pltpu.sync_copy(acc_sh.at[pl.ds(sid * RPS, RPS)],
                        out_hbm.at[pl.ds(cid * NP + sid * RPS, RPS)])

    return k(u, srcb, dstb, zeros128)


_R = 2048
_G = NP // _R


def _tc_prep_body(d0_ref, d1_ref, x_ref, u1_ref, dinv_ref):
    deg = d0_ref[:, :1] + d1_ref[:, :1] + 1.0
    dinv = lax.rsqrt(jnp.maximum(deg, 1e-12))
    dinv_ref[...] = dinv
    u1_ref[...] = dinv * x_ref[...]


def _tc_prep(degp, x_pad):
    return pl.pallas_call(
        _tc_prep_body,
        grid=(_G,),
        in_specs=[
            pl.BlockSpec((_R, D), lambda i: (i, 0)),
            pl.BlockSpec((_R, D), lambda i: (i + NP // _R, 0)),
            pl.BlockSpec((_R, D), lambda i: (i, 0)),
        ],
        out_specs=[
            pl.BlockSpec((_R, D), lambda i: (i, 0)),
            pl.BlockSpec((_R, 1), lambda i: (i, 0)),
        ],
        out_shape=[_f32(NP, D), _f32(NP, 1)],
    )(degp, degp, x_pad)


def _tc_mid_body(s0_ref, s1_ref, x_ref, dinv_ref, w1_ref, b1_ref, w2_ref,
                 t_ref, u2_ref):
    dinv = dinv_ref[...]
    agg1 = dinv * (s0_ref[...] + s1_ref[...]) + (dinv * dinv) * x_ref[...]
    h1 = jnp.maximum(
        jnp.dot(agg1, w1_ref[...], preferred_element_type=jnp.float32)
        + b1_ref[...], 0.0)
    t = jnp.dot(h1, w2_ref[...], preferred_element_type=jnp.float32)
    t_ref[...] = t
    u2_ref[...] = dinv * t


def _tc_mid(s1, x_pad, dinv, W1, b1, W2):
    return pl.pallas_call(
        _tc_mid_body,
        grid=(_G,),
        in_specs=[
            pl.BlockSpec((_R, D), lambda i: (i, 0)),
            pl.BlockSpec((_R, D), lambda i: (i + NP // _R, 0)),
            pl.BlockSpec((_R, D), lambda i: (i, 0)),
            pl.BlockSpec((_R, 1), lambda i: (i, 0)),
            pl.BlockSpec((D, DH), lambda i: (0, 0)),
            pl.BlockSpec((1, DH), lambda i: (0, 0)),
            pl.BlockSpec((DH, D), lambda i: (0, 0)),
        ],
        out_specs=[
            pl.BlockSpec((_R, D), lambda i: (i, 0)),
            pl.BlockSpec((_R, D), lambda i: (i, 0)),
        ],
        out_shape=[_f32(NP, D), _f32(NP, D)],
    )(s1, s1, x_pad, dinv, W1, b1, W2)


def _tc_final_body(s0_ref, s1_ref, t_ref, dinv_ref, b2_ref, out_ref):
    dinv = dinv_ref[...]
    out_ref[...] = (dinv * (s0_ref[...] + s1_ref[...])
                    + (dinv * dinv) * t_ref[...] + b2_ref[...])


def _tc_final(s2, t, dinv, b2):
    return pl.pallas_call(
        _tc_final_body,
        grid=(_G,),
        in_specs=[
            pl.BlockSpec((_R, D), lambda i: (i, 0)),
            pl.BlockSpec((_R, D), lambda i: (i + NP // _R, 0)),
            pl.BlockSpec((_R, D), lambda i: (i, 0)),
            pl.BlockSpec((_R, 1), lambda i: (i, 0)),
            pl.BlockSpec((1, D), lambda i: (0, 0)),
        ],
        out_specs=pl.BlockSpec((_R, D), lambda i: (i, 0)),
        out_shape=_f32(NP, D),
    )(s2, s2, t, dinv, b2)


def kernel(x, edge_index, W1, b1, W2, b2):
    src = edge_index[0].astype(jnp.int32)
    dst = edge_index[1].astype(jnp.int32)
    pad = EP - E
    srcb = jnp.concatenate([src, jnp.zeros((pad,), jnp.int32)]).reshape(NB_ROWS, B)
    dstb = jnp.concatenate([dst, jnp.full((pad,), N, jnp.int32)]).reshape(NB_ROWS, B)
    x_pad = jnp.concatenate([x, jnp.zeros((NP - N, D), x.dtype)], axis=0)

    ones128 = jnp.ones((B, D), jnp.float32)
    zeros128 = jnp.zeros((NP, D), jnp.float32)

    degp = _sc_degree(dstb, ones128, zeros128)
    u1, dinv = _tc_prep(degp, x_pad)
    s1 = _sc_aggregate(u1, srcb, dstb, zeros128)
    t, u2 = _tc_mid(s1, x_pad, dinv, W1, b1.reshape(1, DH), W2)
    s2 = _sc_aggregate(u2, srcb, dstb, zeros128)
    out = _tc_final(s2, t, dinv, b2.reshape(1, D))
    return out[:N]


# speedup vs baseline: 1.4483x; 1.4483x over previous
import dataclasses
import functools

import jax
import jax.numpy as jnp
from jax import lax
from jax.experimental import pallas as pl
from jax.experimental.pallas import tpu as pltpu
from jax.experimental.pallas import tpu_sc as plsc

N = 10000
NP = 10240
D = 128
DH = 256
E = 320000
NC, NS = 2, 16
NW = NC * NS
B = 128
EP = 327680
JB = EP // (NW * B)
NB_ROWS = EP // B
RPS = NP // NS

CH = 16
JB0 = 144
JB1 = 16

def _mesh_fn():
    return plsc.VectorSubcoreMesh(core_axis_name="c", subcore_axis_name="s")


def _f32(*shape):
    return jax.ShapeDtypeStruct(shape, jnp.float32)


L = 16


def _sc_degree(dstb):
    cp = pltpu.CompilerParams()
    if "needs_layout_passes" in pltpu.CompilerParams.__dataclass_fields__:
        cp = dataclasses.replace(cp, needs_layout_passes=False)

    @functools.partial(
        pl.kernel,
        out_type=_f32(NC * NP),
        mesh=_mesh_fn(),
        compiler_params=cp,
        scratch_types=[
            pltpu.VMEM((JB, B), jnp.int32),
            pltpu.VMEM((NP,), jnp.float32),
            pltpu.VMEM((NS, RPS), jnp.float32),
            pltpu.VMEM((RPS,), jnp.float32),
            pltpu.VMEM_SHARED((NS, NP), jnp.float32),
        ],
    )
    def k(dst_hbm, out_hbm, idx_v, hist, buf, res, stage_sh):
        cid = lax.axis_index("c")
        sid = lax.axis_index("s")
        w = cid * NS + sid
        pltpu.sync_copy(dst_hbm.at[pl.ds(w * JB, JB)], idx_v)

        zeros = jnp.zeros((L,), jnp.float32)
        ones = jnp.ones((L,), jnp.float32)

        @pl.loop(0, NP, step=L)
        def _(i):
            hist[pl.ds(i, L)] = zeros

        @pl.loop(0, JB)
        def _(j):
            @pl.loop(0, B, step=L)
            def _(c):
                plsc.addupdate_scatter(hist, [idx_v[j, pl.ds(c, L)]], ones)

        pltpu.sync_copy(hist, stage_sh.at[sid])
        plsc.subcore_barrier()
        pltpu.sync_copy(stage_sh.at[:, pl.ds(sid * RPS, RPS)], buf)

        @pl.loop(0, RPS, step=L)
        def _(i):
            acc = buf[0, pl.ds(i, L)]
            for r in range(1, NS):
                acc = acc + buf[r, pl.ds(i, L)]
            res[pl.ds(i, L)] = acc

        pltpu.sync_copy(res, out_hbm.at[pl.ds(cid * NP + sid * RPS, RPS)])

    return k(dstb)


NBUF = 2


def _sc_aggregate(u, srcb, dstb, zeros128):
    @functools.partial(
        pl.kernel,
        out_type=_f32(NC * NP, D),
        mesh=_mesh_fn(),
        scratch_types=[
            pltpu.VMEM((CH, B), jnp.int32),
            pltpu.VMEM((CH, B), jnp.int32),
        ] + [pltpu.VMEM((B, D), jnp.float32)] * NBUF
          + [pltpu.VMEM_SHARED((NP, D), jnp.float32)]
          + [pltpu.SemaphoreType.DMA] * NBUF,
    )
    def k(u_hbm, src_hbm, dst_hbm, zeros_hbm, out_hbm,
          srcv, dstv, *rest):
        rows = rest[:NBUF]
        acc_sh = rest[NBUF]
        gsem = rest[NBUF + 1:]
        cid = lax.axis_index("c")
        sid = lax.axis_index("s")

        pltpu.sync_copy(zeros_hbm.at[pl.ds(sid * RPS, RPS)],
                        acc_sh.at[pl.ds(sid * RPS, RPS)])
        plsc.subcore_barrier()

        def process_chunk(base):
            pltpu.sync_copy(src_hbm.at[pl.ds(base, CH)], srcv)
            pltpu.sync_copy(dst_hbm.at[pl.ds(base, CH)], dstv)

            for b in range(NBUF):
                pltpu.async_copy(u_hbm.at[srcv.at[b]], rows[b], gsem[b])

            @pl.loop(0, CH - NBUF, step=NBUF)
            def _(g):
                for b in range(NBUF):
                    j = g + b
                    pltpu.make_async_copy(u_hbm.at[srcv.at[j]],
                                          rows[b], gsem[b]).wait()
                    pltpu.sync_copy(rows[b], acc_sh.at[dstv.at[j]], add=True)
                    pltpu.async_copy(u_hbm.at[srcv.at[j + NBUF]],
                                     rows[b], gsem[b])

            for b in range(NBUF):
                j = CH - NBUF + b
                pltpu.make_async_copy(u_hbm.at[srcv.at[j]],
                                      rows[b], gsem[b]).wait()
                pltpu.sync_copy(rows[b], acc_sh.at[dstv.at[j]], add=True)

        @pl.when(cid == 0)
        def _():
            for c in range(JB0 // CH):
                process_chunk(sid * JB0 + c * CH)

        @pl.when(cid == 1)
        def _():
            for c in range(JB1 // CH):
                process_chunk(NS * JB0 + sid * JB1 + c * CH)

        plsc.subcore_barrier()
        pltpu.sync_copy(acc_sh.at[pl.ds(sid * RPS, RPS)],
                        out_hbm.at[pl.ds(cid * NP + sid * RPS, RPS)])

    return k(u, srcb, dstb, zeros128)


_R = 2048
_G = NP // _R


def _tc_prep_body(d0_ref, d1_ref, x_ref, u1_ref, dinv_ref):
    deg = d0_ref[...] + d1_ref[...] + 1.0
    dinv = lax.rsqrt(jnp.maximum(deg, 1e-12))
    dinv_ref[...] = dinv
    u1_ref[...] = dinv * x_ref[...]


def _tc_prep(d0, d1, x_pad):
    return pl.pallas_call(
        _tc_prep_body,
        grid=(_G,),
        in_specs=[
            pl.BlockSpec((_R, 1), lambda i: (i, 0)),
            pl.BlockSpec((_R, 1), lambda i: (i, 0)),
            pl.BlockSpec((_R, D), lambda i: (i, 0)),
        ],
        out_specs=[
            pl.BlockSpec((_R, D), lambda i: (i, 0)),
            pl.BlockSpec((_R, 1), lambda i: (i, 0)),
        ],
        out_shape=[_f32(NP, D), _f32(NP, 1)],
    )(d0, d1, x_pad)


def _tc_mid_body(s0_ref, s1_ref, x_ref, dinv_ref, w1_ref, b1_ref, w2_ref,
                 t_ref, u2_ref):
    dinv = dinv_ref[...]
    agg1 = dinv * (s0_ref[...] + s1_ref[...]) + (dinv * dinv) * x_ref[...]
    h1 = jnp.maximum(
        jnp.dot(agg1, w1_ref[...], preferred_element_type=jnp.float32)
        + b1_ref[...], 0.0)
    t = jnp.dot(h1, w2_ref[...], preferred_element_type=jnp.float32)
    t_ref[...] = t
    u2_ref[...] = dinv * t


def _tc_mid(s1, x_pad, dinv, W1, b1, W2):
    return pl.pallas_call(
        _tc_mid_body,
        grid=(_G,),
        in_specs=[
            pl.BlockSpec((_R, D), lambda i: (i, 0)),
            pl.BlockSpec((_R, D), lambda i: (i + NP // _R, 0)),
            pl.BlockSpec((_R, D), lambda i: (i, 0)),
            pl.BlockSpec((_R, 1), lambda i: (i, 0)),
            pl.BlockSpec((D, DH), lambda i: (0, 0)),
            pl.BlockSpec((1, DH), lambda i: (0, 0)),
            pl.BlockSpec((DH, D), lambda i: (0, 0)),
        ],
        out_specs=[
            pl.BlockSpec((_R, D), lambda i: (i, 0)),
            pl.BlockSpec((_R, D), lambda i: (i, 0)),
        ],
        out_shape=[_f32(NP, D), _f32(NP, D)],
    )(s1, s1, x_pad, dinv, W1, b1, W2)


def _tc_final_body(s0_ref, s1_ref, t_ref, dinv_ref, b2_ref, out_ref):
    dinv = dinv_ref[...]
    out_ref[...] = (dinv * (s0_ref[...] + s1_ref[...])
                    + (dinv * dinv) * t_ref[...] + b2_ref[...])


def _tc_final(s2, t, dinv, b2):
    return pl.pallas_call(
        _tc_final_body,
        grid=(_G,),
        in_specs=[
            pl.BlockSpec((_R, D), lambda i: (i, 0)),
            pl.BlockSpec((_R, D), lambda i: (i + NP // _R, 0)),
            pl.BlockSpec((_R, D), lambda i: (i, 0)),
            pl.BlockSpec((_R, 1), lambda i: (i, 0)),
            pl.BlockSpec((1, D), lambda i: (0, 0)),
        ],
        out_specs=pl.BlockSpec((_R, D), lambda i: (i, 0)),
        out_shape=_f32(NP, D),
    )(s2, s2, t, dinv, b2)


def kernel(x, edge_index, W1, b1, W2, b2):
    src = edge_index[0].astype(jnp.int32)
    dst = edge_index[1].astype(jnp.int32)
    pad = EP - E
    srcb = jnp.concatenate([src, jnp.zeros((pad,), jnp.int32)]).reshape(NB_ROWS, B)
    dstb = jnp.concatenate([dst, jnp.full((pad,), N, jnp.int32)]).reshape(NB_ROWS, B)
    x_pad = jnp.concatenate([x, jnp.zeros((NP - N, D), x.dtype)], axis=0)

    zeros128 = jnp.zeros((NP, D), jnp.float32)

    degp = _sc_degree(dstb)
    u1, dinv = _tc_prep(degp[:NP].reshape(NP, 1), degp[NP:].reshape(NP, 1),
                        x_pad)
    s1 = _sc_aggregate(u1, srcb, dstb, zeros128)
    t, u2 = _tc_mid(s1, x_pad, dinv, W1, b1.reshape(1, DH), W2)
    s2 = _sc_aggregate(u2, srcb, dstb, zeros128)
    out = _tc_final(s2, t, dinv, b2.reshape(1, D))
    return out[:N]


# speedup vs baseline: 3.7948x; 2.6201x over previous
import dataclasses
import functools

import jax
import jax.numpy as jnp
from jax import lax
from jax.experimental import pallas as pl
from jax.experimental.pallas import tpu as pltpu
from jax.experimental.pallas import tpu_sc as plsc

N = 10000
NP = 10240
D = 128
DH = 256
E = 320000
NC, NS = 2, 16
NW = NC * NS
B = 128
EP = 327680
JB = EP // (NW * B)
NB_ROWS = EP // B
RPS = NP // NS

CH = 16
JB0 = 80
JB1 = 80

def _mesh_fn():
    return plsc.VectorSubcoreMesh(core_axis_name="c", subcore_axis_name="s")


def _f32(*shape):
    return jax.ShapeDtypeStruct(shape, jnp.float32)


L = 16


def _sc_degree(dstb):
    cp = pltpu.CompilerParams()
    if "needs_layout_passes" in pltpu.CompilerParams.__dataclass_fields__:
        cp = dataclasses.replace(cp, needs_layout_passes=False)

    @functools.partial(
        pl.kernel,
        out_type=_f32(NC * NP),
        mesh=_mesh_fn(),
        compiler_params=cp,
        scratch_types=[
            pltpu.VMEM((JB, B), jnp.int32),
            pltpu.VMEM((NP,), jnp.float32),
            pltpu.VMEM((NS, RPS), jnp.float32),
            pltpu.VMEM((RPS,), jnp.float32),
            pltpu.VMEM_SHARED((NS, NP), jnp.float32),
        ],
    )
    def k(dst_hbm, out_hbm, idx_v, hist, buf, res, stage_sh):
        cid = lax.axis_index("c")
        sid = lax.axis_index("s")
        w = cid * NS + sid
        pltpu.sync_copy(dst_hbm.at[pl.ds(w * JB, JB)], idx_v)

        zeros = jnp.zeros((L,), jnp.float32)
        ones = jnp.ones((L,), jnp.float32)

        @pl.loop(0, NP, step=L)
        def _(i):
            hist[pl.ds(i, L)] = zeros

        @pl.loop(0, JB)
        def _(j):
            @pl.loop(0, B, step=L)
            def _(c):
                plsc.addupdate_scatter(hist, [idx_v[j, pl.ds(c, L)]], ones)

        pltpu.sync_copy(hist, stage_sh.at[sid])
        plsc.subcore_barrier()
        pltpu.sync_copy(stage_sh.at[:, pl.ds(sid * RPS, RPS)], buf)

        @pl.loop(0, RPS, step=L)
        def _(i):
            acc = buf[0, pl.ds(i, L)]
            for r in range(1, NS):
                acc = acc + buf[r, pl.ds(i, L)]
            res[pl.ds(i, L)] = acc

        pltpu.sync_copy(res, out_hbm.at[pl.ds(cid * NP + sid * RPS, RPS)])

    return k(dstb)


NBUF = 2


def _sc_aggregate(u, srcb, dstb, zeros128):
    @functools.partial(
        pl.kernel,
        out_type=_f32(NC * NP, D),
        mesh=_mesh_fn(),
        scratch_types=[
            pltpu.VMEM((CH, B), jnp.int32),
            pltpu.VMEM((CH, B), jnp.int32),
        ] + [pltpu.VMEM((B, D), jnp.float32)] * NBUF
          + [pltpu.VMEM_SHARED((NP, D), jnp.float32)]
          + [pltpu.SemaphoreType.DMA] * NBUF,
    )
    def k(u_hbm, src_hbm, dst_hbm, zeros_hbm, out_hbm,
          srcv, dstv, *rest):
        rows = rest[:NBUF]
        acc_sh = rest[NBUF]
        gsem = rest[NBUF + 1:]
        cid = lax.axis_index("c")
        sid = lax.axis_index("s")

        pltpu.sync_copy(zeros_hbm.at[pl.ds(sid * RPS, RPS)],
                        acc_sh.at[pl.ds(sid * RPS, RPS)])
        plsc.subcore_barrier()

        def process_chunk(base):
            pltpu.sync_copy(src_hbm.at[pl.ds(base, CH)], srcv)
            pltpu.sync_copy(dst_hbm.at[pl.ds(base, CH)], dstv)

            for b in range(NBUF):
                pltpu.async_copy(u_hbm.at[srcv.at[b]], rows[b], gsem[b])

            @pl.loop(0, CH - NBUF, step=NBUF)
            def _(g):
                for b in range(NBUF):
                    j = g + b
                    pltpu.make_async_copy(u_hbm.at[srcv.at[j]],
                                          rows[b], gsem[b]).wait()
                    pltpu.sync_copy(rows[b], acc_sh.at[dstv.at[j]], add=True)
                    pltpu.async_copy(u_hbm.at[srcv.at[j + NBUF]],
                                     rows[b], gsem[b])

            for b in range(NBUF):
                j = CH - NBUF + b
                pltpu.make_async_copy(u_hbm.at[srcv.at[j]],
                                      rows[b], gsem[b]).wait()
                pltpu.sync_copy(rows[b], acc_sh.at[dstv.at[j]], add=True)

        @pl.when(cid == 0)
        def _():
            for c in range(JB0 // CH):
                process_chunk(sid * JB0 + c * CH)

        @pl.when(cid == 1)
        def _():
            for c in range(JB1 // CH):
                process_chunk(NS * JB0 + sid * JB1 + c * CH)

        plsc.subcore_barrier()
        pltpu.sync_copy(acc_sh.at[pl.ds(sid * RPS, RPS)],
                        out_hbm.at[pl.ds(cid * NP + sid * RPS, RPS)])

    return k(u, srcb, dstb, zeros128)


_R = 2048
_G = NP // _R


def _tc_prep_body(d0_ref, d1_ref, x_ref, u1_ref, dinv_ref):
    deg = d0_ref[...] + d1_ref[...] + 1.0
    dinv = lax.rsqrt(jnp.maximum(deg, 1e-12))
    dinv_ref[...] = dinv
    u1_ref[...] = dinv * x_ref[...]


def _tc_prep(d0, d1, x_pad):
    return pl.pallas_call(
        _tc_prep_body,
        grid=(_G,),
        in_specs=[
            pl.BlockSpec((_R, 1), lambda i: (i, 0)),
            pl.BlockSpec((_R, 1), lambda i: (i, 0)),
            pl.BlockSpec((_R, D), lambda i: (i, 0)),
        ],
        out_specs=[
            pl.BlockSpec((_R, D), lambda i: (i, 0)),
            pl.BlockSpec((_R, 1), lambda i: (i, 0)),
        ],
        out_shape=[_f32(NP, D), _f32(NP, 1)],
    )(d0, d1, x_pad)


def _tc_mid_body(s0_ref, s1_ref, x_ref, dinv_ref, w1_ref, b1_ref, w2_ref,
                 t_ref, u2_ref):
    dinv = dinv_ref[...]
    agg1 = dinv * (s0_ref[...] + s1_ref[...]) + (dinv * dinv) * x_ref[...]
    h1 = jnp.maximum(
        jnp.dot(agg1, w1_ref[...], preferred_element_type=jnp.float32)
        + b1_ref[...], 0.0)
    t = jnp.dot(h1, w2_ref[...], preferred_element_type=jnp.float32)
    t_ref[...] = t
    u2_ref[...] = dinv * t


def _tc_mid(s1, x_pad, dinv, W1, b1, W2):
    return pl.pallas_call(
        _tc_mid_body,
        grid=(_G,),
        in_specs=[
            pl.BlockSpec((_R, D), lambda i: (i, 0)),
            pl.BlockSpec((_R, D), lambda i: (i + NP // _R, 0)),
            pl.BlockSpec((_R, D), lambda i: (i, 0)),
            pl.BlockSpec((_R, 1), lambda i: (i, 0)),
            pl.BlockSpec((D, DH), lambda i: (0, 0)),
            pl.BlockSpec((1, DH), lambda i: (0, 0)),
            pl.BlockSpec((DH, D), lambda i: (0, 0)),
        ],
        out_specs=[
            pl.BlockSpec((_R, D), lambda i: (i, 0)),
            pl.BlockSpec((_R, D), lambda i: (i, 0)),
        ],
        out_shape=[_f32(NP, D), _f32(NP, D)],
    )(s1, s1, x_pad, dinv, W1, b1, W2)


def _tc_final_body(s0_ref, s1_ref, t_ref, dinv_ref, b2_ref, out_ref):
    dinv = dinv_ref[...]
    out_ref[...] = (dinv * (s0_ref[...] + s1_ref[...])
                    + (dinv * dinv) * t_ref[...] + b2_ref[...])


def _tc_final(s2, t, dinv, b2):
    return pl.pallas_call(
        _tc_final_body,
        grid=(_G,),
        in_specs=[
            pl.BlockSpec((_R, D), lambda i: (i, 0)),
            pl.BlockSpec((_R, D), lambda i: (i + NP // _R, 0)),
            pl.BlockSpec((_R, D), lambda i: (i, 0)),
            pl.BlockSpec((_R, 1), lambda i: (i, 0)),
            pl.BlockSpec((1, D), lambda i: (0, 0)),
        ],
        out_specs=pl.BlockSpec((_R, D), lambda i: (i, 0)),
        out_shape=_f32(NP, D),
    )(s2, s2, t, dinv, b2)


def kernel(x, edge_index, W1, b1, W2, b2):
    src = edge_index[0].astype(jnp.int32)
    dst = edge_index[1].astype(jnp.int32)
    pad = EP - E
    pad_idx = (N + jnp.arange(pad, dtype=jnp.int32) % (NP - N))
    srcb = jnp.concatenate([src, pad_idx]).reshape(NB_ROWS, B)
    dstb = jnp.concatenate([dst, pad_idx]).reshape(NB_ROWS, B)
    x_pad = jnp.concatenate([x, jnp.zeros((NP - N, D), x.dtype)], axis=0)

    zeros128 = jnp.zeros((NP, D), jnp.float32)

    degp = _sc_degree(dstb)
    u1, dinv = _tc_prep(degp[:NP].reshape(NP, 1), degp[NP:].reshape(NP, 1),
                        x_pad)
    s1 = _sc_aggregate(u1, srcb, dstb, zeros128)
    t, u2 = _tc_mid(s1, x_pad, dinv, W1, b1.reshape(1, DH), W2)
    s2 = _sc_aggregate(u2, srcb, dstb, zeros128)
    out = _tc_final(s2, t, dinv, b2.reshape(1, D))
    return out[:N]


# speedup vs baseline: 4.0102x; 1.0568x over previous
import dataclasses
import functools

import jax
import jax.numpy as jnp
from jax import lax
from jax.experimental import pallas as pl
from jax.experimental.pallas import tpu as pltpu
from jax.experimental.pallas import tpu_sc as plsc

N = 10000
NP = 10240
D = 128
DH = 256
E = 320000
NC, NS = 2, 16
NW = NC * NS
B = 128
EP = 327680
JB = EP // (NW * B)
NB_ROWS = EP // B
RPS = NP // NS

CH = 40
JB0 = 80
JB1 = 80

def _mesh_fn():
    return plsc.VectorSubcoreMesh(core_axis_name="c", subcore_axis_name="s")


def _f32(*shape):
    return jax.ShapeDtypeStruct(shape, jnp.float32)


L = 16


def _sc_degree(dstb):
    cp = pltpu.CompilerParams()
    if "needs_layout_passes" in pltpu.CompilerParams.__dataclass_fields__:
        cp = dataclasses.replace(cp, needs_layout_passes=False)

    @functools.partial(
        pl.kernel,
        out_type=_f32(NC * NP),
        mesh=_mesh_fn(),
        compiler_params=cp,
        scratch_types=[
            pltpu.VMEM((JB, B), jnp.int32),
            pltpu.VMEM((NP,), jnp.float32),
            pltpu.VMEM((NS, RPS), jnp.float32),
            pltpu.VMEM((RPS,), jnp.float32),
            pltpu.VMEM_SHARED((NS, NP), jnp.float32),
        ],
    )
    def k(dst_hbm, out_hbm, idx_v, hist, buf, res, stage_sh):
        cid = lax.axis_index("c")
        sid = lax.axis_index("s")
        w = cid * NS + sid
        pltpu.sync_copy(dst_hbm.at[pl.ds(w * JB, JB)], idx_v)

        zeros = jnp.zeros((L,), jnp.float32)
        ones = jnp.ones((L,), jnp.float32)

        @pl.loop(0, NP, step=L)
        def _(i):
            hist[pl.ds(i, L)] = zeros

        @pl.loop(0, JB)
        def _(j):
            @pl.loop(0, B, step=L)
            def _(c):
                plsc.addupdate_scatter(hist, [idx_v[j, pl.ds(c, L)]], ones)

        pltpu.sync_copy(hist, stage_sh.at[sid])
        plsc.subcore_barrier()
        pltpu.sync_copy(stage_sh.at[:, pl.ds(sid * RPS, RPS)], buf)

        @pl.loop(0, RPS, step=L)
        def _(i):
            acc = buf[0, pl.ds(i, L)]
            for r in range(1, NS):
                acc = acc + buf[r, pl.ds(i, L)]
            res[pl.ds(i, L)] = acc

        pltpu.sync_copy(res, out_hbm.at[pl.ds(cid * NP + sid * RPS, RPS)])

    return k(dstb)


NBUF = 2


def _sc_aggregate(u, srcb, dstb, zeros128):
    @functools.partial(
        pl.kernel,
        out_type=_f32(NC * NP, D),
        mesh=_mesh_fn(),
        scratch_types=[
            pltpu.VMEM((CH, B), jnp.int32),
            pltpu.VMEM((CH, B), jnp.int32),
        ] + [pltpu.VMEM((B, D), jnp.float32)] * NBUF
          + [pltpu.VMEM_SHARED((NP, D), jnp.float32)]
          + [pltpu.SemaphoreType.DMA] * NBUF,
    )
    def k(u_hbm, src_hbm, dst_hbm, zeros_hbm, out_hbm,
          srcv, dstv, *rest):
        rows = rest[:NBUF]
        acc_sh = rest[NBUF]
        gsem = rest[NBUF + 1:]
        cid = lax.axis_index("c")
        sid = lax.axis_index("s")

        pltpu.sync_copy(zeros_hbm.at[pl.ds(sid * RPS, RPS)],
                        acc_sh.at[pl.ds(sid * RPS, RPS)])
        plsc.subcore_barrier()

        def process_chunk(base):
            pltpu.sync_copy(src_hbm.at[pl.ds(base, CH)], srcv)
            pltpu.sync_copy(dst_hbm.at[pl.ds(base, CH)], dstv)

            for b in range(NBUF):
                pltpu.async_copy(u_hbm.at[srcv.at[b]], rows[b], gsem[b])

            @pl.loop(0, CH - NBUF, step=NBUF)
            def _(g):
                for b in range(NBUF):
                    j = g + b
                    pltpu.make_async_copy(u_hbm.at[srcv.at[j]],
                                          rows[b], gsem[b]).wait()
                    pltpu.sync_copy(rows[b], acc_sh.at[dstv.at[j]], add=True)
                    pltpu.async_copy(u_hbm.at[srcv.at[j + NBUF]],
                                     rows[b], gsem[b])

            for b in range(NBUF):
                j = CH - NBUF + b
                pltpu.make_async_copy(u_hbm.at[srcv.at[j]],
                                      rows[b], gsem[b]).wait()
                pltpu.sync_copy(rows[b], acc_sh.at[dstv.at[j]], add=True)

        @pl.when(cid == 0)
        def _():
            for c in range(JB0 // CH):
                process_chunk(sid * JB0 + c * CH)

        @pl.when(cid == 1)
        def _():
            for c in range(JB1 // CH):
                process_chunk(NS * JB0 + sid * JB1 + c * CH)

        plsc.subcore_barrier()
        pltpu.sync_copy(acc_sh.at[pl.ds(sid * RPS, RPS)],
                        out_hbm.at[pl.ds(cid * NP + sid * RPS, RPS)])

    return k(u, srcb, dstb, zeros128)


_R = 2048
_G = NP // _R


def _tc_prep_body(d0_ref, d1_ref, x_ref, u1_ref, dinv_ref):
    deg = d0_ref[...] + d1_ref[...] + 1.0
    dinv = lax.rsqrt(jnp.maximum(deg, 1e-12))
    dinv_ref[...] = dinv
    u1_ref[...] = dinv * x_ref[...]


def _tc_prep(d0, d1, x_pad):
    return pl.pallas_call(
        _tc_prep_body,
        grid=(_G,),
        in_specs=[
            pl.BlockSpec((_R, 1), lambda i: (i, 0)),
            pl.BlockSpec((_R, 1), lambda i: (i, 0)),
            pl.BlockSpec((_R, D), lambda i: (i, 0)),
        ],
        out_specs=[
            pl.BlockSpec((_R, D), lambda i: (i, 0)),
            pl.BlockSpec((_R, 1), lambda i: (i, 0)),
        ],
        out_shape=[_f32(NP, D), _f32(NP, 1)],
    )(d0, d1, x_pad)


def _tc_mid_body(s0_ref, s1_ref, x_ref, dinv_ref, w1_ref, b1_ref, w2_ref,
                 t_ref, u2_ref):
    dinv = dinv_ref[...]
    agg1 = dinv * (s0_ref[...] + s1_ref[...]) + (dinv * dinv) * x_ref[...]
    h1 = jnp.maximum(
        jnp.dot(agg1, w1_ref[...], preferred_element_type=jnp.float32)
        + b1_ref[...], 0.0)
    t = jnp.dot(h1, w2_ref[...], preferred_element_type=jnp.float32)
    t_ref[...] = t
    u2_ref[...] = dinv * t


def _tc_mid(s1, x_pad, dinv, W1, b1, W2):
    return pl.pallas_call(
        _tc_mid_body,
        grid=(_G,),
        in_specs=[
            pl.BlockSpec((_R, D), lambda i: (i, 0)),
            pl.BlockSpec((_R, D), lambda i: (i + NP // _R, 0)),
            pl.BlockSpec((_R, D), lambda i: (i, 0)),
            pl.BlockSpec((_R, 1), lambda i: (i, 0)),
            pl.BlockSpec((D, DH), lambda i: (0, 0)),
            pl.BlockSpec((1, DH), lambda i: (0, 0)),
            pl.BlockSpec((DH, D), lambda i: (0, 0)),
        ],
        out_specs=[
            pl.BlockSpec((_R, D), lambda i: (i, 0)),
            pl.BlockSpec((_R, D), lambda i: (i, 0)),
        ],
        out_shape=[_f32(NP, D), _f32(NP, D)],
    )(s1, s1, x_pad, dinv, W1, b1, W2)


def _tc_final_body(s0_ref, s1_ref, t_ref, dinv_ref, b2_ref, out_ref):
    dinv = dinv_ref[...]
    out_ref[...] = (dinv * (s0_ref[...] + s1_ref[...])
                    + (dinv * dinv) * t_ref[...] + b2_ref[...])


def _tc_final(s2, t, dinv, b2):
    return pl.pallas_call(
        _tc_final_body,
        grid=(_G,),
        in_specs=[
            pl.BlockSpec((_R, D), lambda i: (i, 0)),
            pl.BlockSpec((_R, D), lambda i: (i + NP // _R, 0)),
            pl.BlockSpec((_R, D), lambda i: (i, 0)),
            pl.BlockSpec((_R, 1), lambda i: (i, 0)),
            pl.BlockSpec((1, D), lambda i: (0, 0)),
        ],
        out_specs=pl.BlockSpec((_R, D), lambda i: (i, 0)),
        out_shape=_f32(NP, D),
    )(s2, s2, t, dinv, b2)


def kernel(x, edge_index, W1, b1, W2, b2):
    src = edge_index[0].astype(jnp.int32)
    dst = edge_index[1].astype(jnp.int32)
    pad = EP - E
    pad_idx = (N + jnp.arange(pad, dtype=jnp.int32) % (NP - N))
    srcb = jnp.concatenate([src, pad_idx]).reshape(NB_ROWS, B)
    dstb = jnp.concatenate([dst, pad_idx]).reshape(NB_ROWS, B)
    x_pad = jnp.concatenate([x, jnp.zeros((NP - N, D), x.dtype)], axis=0)

    zeros128 = jnp.zeros((NP, D), jnp.float32)

    degp = _sc_degree(dstb)
    u1, dinv = _tc_prep(degp[:NP].reshape(NP, 1), degp[NP:].reshape(NP, 1),
                        x_pad)
    s1 = _sc_aggregate(u1, srcb, dstb, zeros128)
    t, u2 = _tc_mid(s1, x_pad, dinv, W1, b1.reshape(1, DH), W2)
    s2 = _sc_aggregate(u2, srcb, dstb, zeros128)
    out = _tc_final(s2, t, dinv, b2.reshape(1, D))
    return out[:N]
